# baseline (speedup 1.0000x reference)
import functools

import jax
import jax.numpy as jnp
from jax import lax
from jax.experimental import pallas as pl
from jax.experimental.pallas import tpu as pltpu
from jax.experimental.pallas import tpu_sc as plsc

BATCH = 16384
HIDDEN = 64
FEAT_DIM = 20
U_ROWS = 1000000
M_ROWS = 100000

_NC = 2
_NS = 16
_NW = _NC * _NS
_BOUNCE = 256
_MCAP = 2048
_HCAP = 144
_CHUNK = 2048
_OUT_ROWS = BATCH + 128


def _scan_table(ids_hbm, tab_hbm, out_hbm, nrows, ntiles, wid,
                idchunk, mat_id, mat_pos, hit_id, hit_pos, bbuf, stag,
                bsems, ssems, ngrp):
    t_lo = (wid * ntiles) >> 5
    t_hi = ((wid + 1) * ntiles) >> 5
    lo = t_lo * 8
    hi = t_hi * 8

    def chunk_body(ci, off):
        pltpu.sync_copy(ids_hbm.at[pl.ds(ci * _CHUNK, _CHUNK)], idchunk)

        def vreg_body(g, off):
            v = idchunk[pl.ds(g * 16, 16)]
            m = (v >= lo) & (v < hi)
            slot = jnp.minimum(off + plsc.cumsum(m.astype(jnp.int32)) - 1,
                               _MCAP - 1)
            plsc.store_scatter(mat_id, [slot], v, mask=m)
            pos = ci * _CHUNK + g * 16 + lax.iota(jnp.int32, 16)
            plsc.store_scatter(mat_pos, [slot], pos, mask=m)
            cnt = plsc.all_reduce_population_count(m)[0]
            return jnp.minimum(off + cnt, _MCAP - 16)

        return lax.fori_loop(0, _CHUNK // 16, vreg_body, off)

    n_mat = lax.fori_loop(0, BATCH // _CHUNK, chunk_body, jnp.int32(0))
    n_grp = (n_mat + 15) >> 4
    nb = (hi - lo + _BOUNCE - 1) >> 8

    def start_bounce(b, par):
        s = lo + b * _BOUNCE
        s_c = jnp.minimum(s, nrows - _BOUNCE)
        pltpu.make_async_copy(
            tab_hbm.at[pl.ds(s_c, _BOUNCE)], bbuf.at[par], bsems[par]
        ).start()

    def wait_bounce(par):
        pltpu.make_async_copy(
            tab_hbm.at[pl.ds(0, _BOUNCE)], bbuf.at[par], bsems[par]
        ).wait()

    def wait_scats(par):
        for grp in range(ngrp):
            pltpu.make_async_copy(
                out_hbm.at[pl.ds(0, 16)],
                stag.at[par].at[pl.ds(grp * 16, 16)],
                ssems[par],
            ).wait()

    def process_bounce(b, par):
        s = lo + b * _BOUNCE
        s_c = jnp.minimum(s, nrows - _BOUNCE)
        we = jnp.minimum(s + _BOUNCE, hi)
        buf = bbuf.at[par]
        st = stag.at[par]

        def ref_body(g, nh):
            idv = mat_id[pl.ds(g * 16, 16)]
            posv = mat_pos[pl.ds(g * 16, 16)]
            valid = (g * 16 + lax.iota(jnp.int32, 16)) < n_mat
            m = valid & (idv >= s) & (idv < we)
            slot = jnp.minimum(nh + plsc.cumsum(m.astype(jnp.int32)) - 1,
                               _HCAP - 1)
            plsc.store_scatter(hit_id, [slot], idv, mask=m)
            plsc.store_scatter(hit_pos, [slot], posv, mask=m)
            cnt = plsc.all_reduce_population_count(m)[0]
            return jnp.minimum(nh + cnt, 16 * ngrp)

        nh = lax.fori_loop(0, n_grp, ref_body, jnp.int32(0))

        for grp in range(ngrp):
            idv = hit_id[pl.ds(grp * 16, 16)]
            posv = hit_pos[pl.ds(grp * 16, 16)]
            lanes = grp * 16 + lax.iota(jnp.int32, 16)
            vmask = lanes < nh
            vmask32 = vmask.astype(jnp.int32)
            outpos = jnp.where(vmask, posv, BATCH + lanes)
            for j in range(16):
                @pl.when(vmask32[j] != 0)
                def _():
                    src = buf.at[idv[j] - s_c]
                    dst = st.at[grp * 16 + j]
                    for k in range(HIDDEN // 16):
                        dst[pl.ds(k * 16, 16)] = src[pl.ds(k * 16, 16)]
            pltpu.make_async_copy(
                st.at[pl.ds(grp * 16, 16)], out_hbm.at[outpos], ssems[par]
            ).start()

    start_bounce(0, 0)

    def pair_body(i, carry):
        b0 = i * 2
        b1 = b0 + 1

        @pl.when(b1 < nb)
        def _():
            start_bounce(b1, 1)

        wait_bounce(0)

        @pl.when(b0 >= 2)
        def _():
            wait_scats(0)

        process_bounce(b0, 0)

        @pl.when(b0 + 2 < nb)
        def _():
            start_bounce(b0 + 2, 0)

        @pl.when(b1 < nb)
        def _():
            wait_bounce(1)

            @pl.when(b1 >= 2)
            def _():
                wait_scats(1)

            process_bounce(b1, 1)

        return carry

    lax.fori_loop(0, (nb + 1) >> 1, pair_body, 0)

    for par in range(2):
        @pl.when((nb >= 1) & (((nb - 1) & 1) == par))
        def _():
            wait_scats(par)

        @pl.when((nb >= 2) & ((nb & 1) == par))
        def _():
            wait_scats(par)


def _sc_body(uid_hbm, mid_hbm, utab_hbm, mtab_hbm, uout_hbm, mout_hbm,
             idchunk, mat_id, mat_pos, hit_id, hit_pos, bbuf, stag,
             bsem0, bsem1, ssem0, ssem1):
    wid = lax.axis_index("s") * _NC + lax.axis_index("c")
    _scan_table(uid_hbm, utab_hbm, uout_hbm, U_ROWS, U_ROWS // 8, wid,
                idchunk, mat_id, mat_pos, hit_id, hit_pos, bbuf, stag,
                (bsem0, bsem1), (ssem0, ssem1), 2)
    _scan_table(mid_hbm, mtab_hbm, mout_hbm, M_ROWS, M_ROWS // 8, wid,
                idchunk, mat_id, mat_pos, hit_id, hit_pos, bbuf, stag,
                (bsem0, bsem1), (ssem0, ssem1), 8)


@functools.cache
def _sc_gather():
    return pl.kernel(
        _sc_body,
        out_type=(
            jax.ShapeDtypeStruct((_OUT_ROWS, 2 * HIDDEN), jnp.float32),
            jax.ShapeDtypeStruct((_OUT_ROWS, 2 * HIDDEN), jnp.float32),
        ),
        mesh=plsc.VectorSubcoreMesh(core_axis_name="c", subcore_axis_name="s"),
        scratch_types=[
            pltpu.VMEM((_CHUNK,), jnp.int32),
            pltpu.VMEM((_MCAP,), jnp.int32),
            pltpu.VMEM((_MCAP,), jnp.int32),
            pltpu.VMEM((_HCAP,), jnp.int32),
            pltpu.VMEM((_HCAP,), jnp.int32),
            pltpu.VMEM((2, _BOUNCE, HIDDEN), jnp.float32),
            pltpu.VMEM((2, 8 * 16, 2 * HIDDEN), jnp.float32),
            pltpu.SemaphoreType.DMA,
            pltpu.SemaphoreType.DMA,
            pltpu.SemaphoreType.DMA,
            pltpu.SemaphoreType.DMA,
        ],
        compiler_params=pltpu.CompilerParams(needs_layout_passes=False),
    )


def _tc_combine_body(feat_ref, us_ref, ms_ref, w_ref, b_ref, out_ref):
    proj = jnp.dot(feat_ref[...], w_ref[...],
                   preferred_element_type=jnp.float32) + b_ref[...]
    u = us_ref[:, :HIDDEN]
    m = ms_ref[:, :HIDDEN]
    out_ref[...] = jnp.sum(u * (m + proj), axis=1).reshape(out_ref.shape)


_TC_ROWS = 2048


def _tc_combine(movie_features, uslab, mslab, W, b2d):
    grid = (BATCH // _TC_ROWS,)
    out = pl.pallas_call(
        _tc_combine_body,
        grid=grid,
        in_specs=[
            pl.BlockSpec((_TC_ROWS, FEAT_DIM), lambda i: (i, 0)),
            pl.BlockSpec((_TC_ROWS, 2 * HIDDEN), lambda i: (i, 0)),
            pl.BlockSpec((_TC_ROWS, 2 * HIDDEN), lambda i: (i, 0)),
            pl.BlockSpec((FEAT_DIM, HIDDEN), lambda i: (0, 0)),
            pl.BlockSpec((1, HIDDEN), lambda i: (0, 0)),
        ],
        out_specs=pl.BlockSpec((_TC_ROWS,), lambda i: (i,)),
        out_shape=jax.ShapeDtypeStruct((BATCH,), jnp.float32),
    )(movie_features, uslab, mslab, W, b2d)
    return out


@jax.jit
def kernel(user_ids, movie_ids, movie_features, user_table, movie_table, W, b):
    uids = user_ids.astype(jnp.int32)
    mids = movie_ids.astype(jnp.int32)
    uslab, mslab = _sc_gather()(uids, mids, user_table, movie_table)
    return _tc_combine(movie_features, uslab, mslab, W, b.reshape(1, HIDDEN))


# speedup vs baseline: 1.6469x; 1.6469x over previous
import functools

import jax
import jax.numpy as jnp
from jax import lax
from jax.experimental import pallas as pl
from jax.experimental.pallas import tpu as pltpu
from jax.experimental.pallas import tpu_sc as plsc

BATCH = 16384
HIDDEN = 64
FEAT_DIM = 20

_NC = 2
_NS = 16
_NW = _NC * _NS
_BPW = BATCH // _NW
_HB = _BPW // 2


def _sc_gather_body(uid_hbm, mid_hbm, utab_hbm, mtab_hbm,
                    uout_hbm, mout_hbm,
                    uid_v, mid_v, slab, gsem, wsem):
    wid = lax.axis_index("s") * _NC + lax.axis_index("c")
    base = wid * _BPW
    pltpu.sync_copy(uid_hbm.at[pl.ds(base, _BPW)], uid_v)
    pltpu.sync_copy(mid_hbm.at[pl.ds(base, _BPW)], mid_v)

    def gather(idx_ref, tab_hbm, half, par):
        pltpu.make_async_copy(
            tab_hbm.at[idx_ref.at[pl.ds(half * _HB, _HB)]], slab.at[par], gsem
        ).start()

    def wait_gather(tab_hbm, par):
        pltpu.make_async_copy(
            tab_hbm.at[pl.ds(0, _HB)], slab.at[par], gsem).wait()

    def write(out_hbm, half, par):
        pltpu.make_async_copy(
            slab.at[par], out_hbm.at[pl.ds(base + half * _HB, _HB)], wsem
        ).start()

    def wait_write(out_hbm, par):
        pltpu.make_async_copy(
            out_hbm.at[pl.ds(0, _HB)], slab.at[par], wsem).wait()

    gather(uid_v, utab_hbm, 0, 0)
    gather(uid_v, utab_hbm, 1, 1)
    wait_gather(utab_hbm, 0)
    write(uout_hbm, 0, 0)
    wait_gather(utab_hbm, 1)
    write(uout_hbm, 1, 1)
    wait_write(uout_hbm, 0)
    gather(mid_v, mtab_hbm, 0, 0)
    wait_write(uout_hbm, 1)
    gather(mid_v, mtab_hbm, 1, 1)
    wait_gather(mtab_hbm, 0)
    write(mout_hbm, 0, 0)
    wait_gather(mtab_hbm, 1)
    write(mout_hbm, 1, 1)
    wait_write(mout_hbm, 0)
    wait_write(mout_hbm, 1)


@functools.cache
def _sc_gather():
    return pl.kernel(
        _sc_gather_body,
        out_type=(
            jax.ShapeDtypeStruct((BATCH, 2 * HIDDEN), jnp.float32),
            jax.ShapeDtypeStruct((BATCH, 2 * HIDDEN), jnp.float32),
        ),
        mesh=plsc.VectorSubcoreMesh(core_axis_name="c", subcore_axis_name="s"),
        scratch_types=[
            pltpu.VMEM((_BPW,), jnp.int32),
            pltpu.VMEM((_BPW,), jnp.int32),
            pltpu.VMEM((2, _HB, 2 * HIDDEN), jnp.float32),
            pltpu.SemaphoreType.DMA,
            pltpu.SemaphoreType.DMA,
        ],
    )


def _tc_combine_body(feat_ref, us_ref, ms_ref, w_ref, b_ref, out_ref):
    proj = jnp.dot(feat_ref[...], w_ref[...],
                   preferred_element_type=jnp.float32) + b_ref[...]
    u = us_ref[:, :HIDDEN]
    m = ms_ref[:, :HIDDEN]
    out_ref[...] = jnp.sum(u * (m + proj), axis=1).reshape(out_ref.shape)


_TC_ROWS = 2048


def _tc_combine(movie_features, uslab, mslab, W, b2d):
    grid = (BATCH // _TC_ROWS,)
    out = pl.pallas_call(
        _tc_combine_body,
        grid=grid,
        in_specs=[
            pl.BlockSpec((_TC_ROWS, FEAT_DIM), lambda i: (i, 0)),
            pl.BlockSpec((_TC_ROWS, 2 * HIDDEN), lambda i: (i, 0)),
            pl.BlockSpec((_TC_ROWS, 2 * HIDDEN), lambda i: (i, 0)),
            pl.BlockSpec((FEAT_DIM, HIDDEN), lambda i: (0, 0)),
            pl.BlockSpec((1, HIDDEN), lambda i: (0, 0)),
        ],
        out_specs=pl.BlockSpec((_TC_ROWS,), lambda i: (i,)),
        out_shape=jax.ShapeDtypeStruct((BATCH,), jnp.float32),
    )(movie_features, uslab, mslab, W, b2d)
    return out


@jax.jit
def kernel(user_ids, movie_ids, movie_features, user_table, movie_table, W, b):
    uids = user_ids.astype(jnp.int32)
    mids = movie_ids.astype(jnp.int32)
    utab_pad = jnp.pad(user_table, ((0, 0), (0, HIDDEN)))
    mtab_pad = jnp.pad(movie_table, ((0, 0), (0, HIDDEN)))
    uslab, mslab = _sc_gather()(uids, mids, utab_pad, mtab_pad)
    return _tc_combine(movie_features, uslab, mslab, W, b.reshape(1, HIDDEN))


# speedup vs baseline: 2.3145x; 1.4053x over previous
import functools

import jax
import jax.numpy as jnp
from jax import lax
from jax.experimental import pallas as pl
from jax.experimental.pallas import tpu as pltpu
from jax.experimental.pallas import tpu_sc as plsc

BATCH = 16384
HIDDEN = 64
FEAT_DIM = 20

_NC = 2
_NS = 16
_NW = _NC * _NS
_BPW = BATCH // _NW
_HB = _BPW // 2


def _sc_gather_body(uid_hbm, mid_hbm, utab_hbm, mtab_hbm,
                    uout_hbm, mout_hbm,
                    ids_v, vrows, srows_all, vsem, ssem, wsem):
    sid = lax.axis_index("s")
    wid = sid * _NC + lax.axis_index("c")
    base = wid * _BPW
    srows = srows_all.at[sid]

    def phase(ids_hbm, tab_hbm, out_hbm, first):
        pltpu.sync_copy(ids_hbm.at[pl.ds(base, _BPW)], ids_v)

        def fire(g, carry):
            vlo = ids_v[pl.ds(g * 16, 16)]
            vhi = ids_v[pl.ds(_HB + g * 16, 16)]
            for j in range(16):
                i = g * 16 + j
                pltpu.make_async_copy(
                    tab_hbm.at[pl.ds(vlo[j], 1)],
                    vrows.at[pl.ds(i, 1)],
                    vsem,
                ).start()
                pltpu.make_async_copy(
                    tab_hbm.at[pl.ds(vhi[j], 1)],
                    srows.at[pl.ds(i, 1)],
                    ssem,
                ).start()
            return carry

        if not first:
            pltpu.make_async_copy(
                out_hbm.at[pl.ds(0, _HB)], vrows, wsem).wait()
            pltpu.make_async_copy(
                out_hbm.at[pl.ds(0, _HB)], srows, wsem).wait()
        lax.fori_loop(0, _HB // 16, fire, 0)
        pltpu.make_async_copy(tab_hbm.at[pl.ds(0, _HB)], vrows, vsem).wait()
        pltpu.make_async_copy(tab_hbm.at[pl.ds(0, _HB)], srows, ssem).wait()
        pltpu.make_async_copy(
            vrows, out_hbm.at[pl.ds(base, _HB)], wsem).start()
        pltpu.make_async_copy(
            srows, out_hbm.at[pl.ds(base + _HB, _HB)], wsem).start()

    phase(uid_hbm, utab_hbm, uout_hbm, True)
    phase(mid_hbm, mtab_hbm, mout_hbm, False)
    pltpu.make_async_copy(mout_hbm.at[pl.ds(0, _HB)], vrows, wsem).wait()
    pltpu.make_async_copy(mout_hbm.at[pl.ds(0, _HB)], srows, wsem).wait()


@functools.cache
def _sc_gather():
    return pl.kernel(
        _sc_gather_body,
        out_type=(
            jax.ShapeDtypeStruct((BATCH, HIDDEN), jnp.float32),
            jax.ShapeDtypeStruct((BATCH, HIDDEN), jnp.float32),
        ),
        mesh=plsc.VectorSubcoreMesh(core_axis_name="c", subcore_axis_name="s"),
        scratch_types=[
            pltpu.VMEM((_BPW,), jnp.int32),
            pltpu.VMEM((_HB, HIDDEN), jnp.float32),
            pltpu.VMEM_SHARED((_NS, _HB, HIDDEN), jnp.float32),
            pltpu.SemaphoreType.DMA,
            pltpu.SemaphoreType.DMA,
            pltpu.SemaphoreType.DMA,
        ],
    )


def _tc_combine_body(feat_ref, u_ref, m_ref, w_ref, b_ref, out_ref):
    proj = jnp.dot(feat_ref[...], w_ref[...],
                   preferred_element_type=jnp.float32) + b_ref[...]
    out_ref[...] = jnp.sum(u_ref[...] * (m_ref[...] + proj),
                           axis=1).reshape(out_ref.shape)


_TC_ROWS = 2048


def _tc_combine(movie_features, user_emb, movie_emb, W, b2d):
    grid = (BATCH // _TC_ROWS,)
    out = pl.pallas_call(
        _tc_combine_body,
        grid=grid,
        in_specs=[
            pl.BlockSpec((_TC_ROWS, FEAT_DIM), lambda i: (i, 0)),
            pl.BlockSpec((_TC_ROWS, HIDDEN), lambda i: (i, 0)),
            pl.BlockSpec((_TC_ROWS, HIDDEN), lambda i: (i, 0)),
            pl.BlockSpec((FEAT_DIM, HIDDEN), lambda i: (0, 0)),
            pl.BlockSpec((1, HIDDEN), lambda i: (0, 0)),
        ],
        out_specs=pl.BlockSpec((_TC_ROWS,), lambda i: (i,)),
        out_shape=jax.ShapeDtypeStruct((BATCH,), jnp.float32),
    )(movie_features, user_emb, movie_emb, W, b2d)
    return out


@jax.jit
def kernel(user_ids, movie_ids, movie_features, user_table, movie_table, W, b):
    uids = user_ids.astype(jnp.int32)
    mids = movie_ids.astype(jnp.int32)
    user_emb, movie_emb = _sc_gather()(uids, mids, user_table, movie_table)
    return _tc_combine(movie_features, user_emb, movie_emb, W,
                       b.reshape(1, HIDDEN))


# speedup vs baseline: 2.4380x; 1.0533x over previous
import functools

import jax
import jax.numpy as jnp
from jax import lax
from jax.experimental import pallas as pl
from jax.experimental.pallas import tpu as pltpu
from jax.experimental.pallas import tpu_sc as plsc

BATCH = 16384
HIDDEN = 64
FEAT_DIM = 20

_NC = 2
_NS = 16
_NW = _NC * _NS
_BPW = BATCH // _NW
_HB = _BPW // 2


def _sc_gather_body(uid_hbm, mid_hbm, utab_hbm, mtab_hbm,
                    uout_hbm, mout_hbm,
                    uid_v, mid_v, rows_v, gsem, wsem):
    wid = lax.axis_index("s") * _NC + lax.axis_index("c")
    base = wid * _BPW
    pltpu.sync_copy(uid_hbm.at[pl.ds(base, _BPW)], uid_v)
    pltpu.sync_copy(mid_hbm.at[pl.ds(base, _BPW)], mid_v)

    def phase(ids_ref, tab_hbm, out_hbm, half, par, first):
        buf = rows_v.at[par]

        def fire(g, carry):
            vec = ids_ref[pl.ds(half * _HB + g * 16, 16)]
            for j in range(16):
                pltpu.make_async_copy(
                    tab_hbm.at[pl.ds(vec[j], 1)],
                    buf.at[pl.ds(g * 16 + j, 1)],
                    gsem,
                ).start()
            return carry

        if not first:
            pltpu.make_async_copy(
                out_hbm.at[pl.ds(0, _HB)], buf, wsem).wait()
        lax.fori_loop(0, _HB // 16, fire, 0)
        pltpu.make_async_copy(tab_hbm.at[pl.ds(0, _HB)], buf, gsem).wait()
        pltpu.make_async_copy(
            buf, out_hbm.at[pl.ds(base + half * _HB, _HB)], wsem
        ).start()

    phase(uid_v, utab_hbm, uout_hbm, 0, 0, True)
    phase(uid_v, utab_hbm, uout_hbm, 1, 1, True)
    phase(mid_v, mtab_hbm, mout_hbm, 0, 0, False)
    phase(mid_v, mtab_hbm, mout_hbm, 1, 1, False)
    pltpu.make_async_copy(mout_hbm.at[pl.ds(0, _HB)], rows_v.at[0], wsem).wait()
    pltpu.make_async_copy(mout_hbm.at[pl.ds(0, _HB)], rows_v.at[1], wsem).wait()


@functools.cache
def _sc_gather():
    return pl.kernel(
        _sc_gather_body,
        out_type=(
            jax.ShapeDtypeStruct((BATCH, HIDDEN), jnp.float32),
            jax.ShapeDtypeStruct((BATCH, HIDDEN), jnp.float32),
        ),
        mesh=plsc.VectorSubcoreMesh(core_axis_name="c", subcore_axis_name="s"),
        scratch_types=[
            pltpu.VMEM((_BPW,), jnp.int32),
            pltpu.VMEM((_BPW,), jnp.int32),
            pltpu.VMEM((2, _HB, HIDDEN), jnp.float32),
            pltpu.SemaphoreType.DMA,
            pltpu.SemaphoreType.DMA,
        ],
    )


def _tc_combine_body(feat_ref, u_ref, m_ref, w_ref, b_ref, out_ref):
    proj = jnp.dot(feat_ref[...], w_ref[...],
                   preferred_element_type=jnp.float32) + b_ref[...]
    out_ref[...] = jnp.sum(u_ref[...] * (m_ref[...] + proj),
                           axis=1).reshape(out_ref.shape)


_TC_ROWS = 2048


def _tc_combine(movie_features, user_emb, movie_emb, W, b2d):
    grid = (BATCH // _TC_ROWS,)
    out = pl.pallas_call(
        _tc_combine_body,
        grid=grid,
        in_specs=[
            pl.BlockSpec((_TC_ROWS, FEAT_DIM), lambda i: (i, 0)),
            pl.BlockSpec((_TC_ROWS, HIDDEN), lambda i: (i, 0)),
            pl.BlockSpec((_TC_ROWS, HIDDEN), lambda i: (i, 0)),
            pl.BlockSpec((FEAT_DIM, HIDDEN), lambda i: (0, 0)),
            pl.BlockSpec((1, HIDDEN), lambda i: (0, 0)),
        ],
        out_specs=pl.BlockSpec((_TC_ROWS,), lambda i: (i,)),
        out_shape=jax.ShapeDtypeStruct((BATCH,), jnp.float32),
    )(movie_features, user_emb, movie_emb, W, b2d)
    return out


@jax.jit
def kernel(user_ids, movie_ids, movie_features, user_table, movie_table, W, b):
    uids = user_ids.astype(jnp.int32)
    mids = movie_ids.astype(jnp.int32)
    user_emb, movie_emb = _sc_gather()(uids, mids, user_table, movie_table)
    return _tc_combine(movie_features, user_emb, movie_emb, W,
                       b.reshape(1, HIDDEN))
